# Initial kernel scaffold; baseline (speedup 1.0000x reference)
import jax
import jax.numpy as jnp
from jax.experimental import pallas as pl


def kernel(x_user, x_video, edge_index_uv, edge_index_vu, W_l_uv, W_r_uv, b_uv, W_l_vu, W_r_vu, b_vu, W_user, b_user, W_video, b_video):
    raise NotImplementedError("write your pallas kernel here")


# speedup vs baseline: 7.5850x; 7.5850x over previous
import functools

import jax
import jax.numpy as jnp
from jax import lax
from jax.experimental import pallas as pl
from jax.experimental.pallas import tpu as pltpu
from jax.experimental.pallas import tpu_sc as plsc

N = 10000
E = 320000
D = 128
H = 64
O = 32

NC = 2
NS = 16
NW = NC * NS
E_W = E // NW
CH = 80
NCHUNK = E_W // CH
N_PAD = 10240
ROWS_W = N_PAD // NS
CNT_L = 16

BL = 1000
NBLK = N // BL


def _pre_body(xu, xv, wlu, wrv, wlv, wru, pu, pv, ru, rv):
    a = xu[...]
    b = xv[...]
    pu[...] = jnp.dot(a, wlu[...], preferred_element_type=jnp.float32)
    ru[...] = jnp.dot(a, wrv[...], preferred_element_type=jnp.float32)
    pv[...] = jnp.dot(b, wlv[...], preferred_element_type=jnp.float32)
    rv[...] = jnp.dot(b, wru[...], preferred_element_type=jnp.float32)


def _pre(x_user, x_video, W_l_uv, W_r_vu, W_l_vu, W_r_uv):
    row = pl.BlockSpec((BL, D), lambda r: (r, 0))
    wsp = pl.BlockSpec((D, H), lambda r: (0, 0))
    out = pl.BlockSpec((BL, H), lambda r: (r, 0))
    f32 = jnp.float32
    return pl.pallas_call(
        _pre_body,
        grid=(NBLK,),
        in_specs=[row, row, wsp, wsp, wsp, wsp],
        out_specs=[out, out, out, out],
        out_shape=[jax.ShapeDtypeStruct((N, H), f32)] * 4,
    )(x_user, x_video, W_l_uv, W_r_vu, W_l_vu, W_r_uv)


def _sc_body(pmsg_u, pmsg_v, src_uv, dst_uv, src_vu, dst_vu,
             sum_v, cnt_v, sum_u, cnt_u,
             src_st, dst_st, srcb, dstb, rows, onesb, zbuf, zbuf16,
             acc, cnt, sem):
    c = lax.axis_index("c")
    s = lax.axis_index("s")
    wid = c * NS + s
    ebase = wid * E_W
    rbase = s * ROWS_W

    zero16 = jnp.zeros((16,), jnp.float32)
    one16 = jnp.full((16,), 1.0, jnp.float32)

    def zrow(i, carry):
        for j in range(H // 16):
            zbuf[i, pl.ds(j * 16, 16)] = zero16
        zbuf16[i, pl.ds(0, 16)] = zero16
        return carry
    lax.fori_loop(0, ROWS_W, zrow, 0)

    def orow(i, carry):
        onesb[i, pl.ds(0, 16)] = one16
        return carry
    lax.fori_loop(0, CH, orow, 0)

    def run_edge_type(src_h, dst_h, table_h, sum_out, cnt_out):
        pltpu.sync_copy(zbuf, acc.at[pl.ds(rbase, ROWS_W)])
        pltpu.sync_copy(zbuf16, cnt.at[pl.ds(rbase, ROWS_W)])
        pltpu.sync_copy(src_h.at[pl.ds(ebase, E_W)], src_st)
        pltpu.sync_copy(dst_h.at[pl.ds(ebase, E_W)], dst_st)
        plsc.subcore_barrier()

        def chunk(k, carry):
            off = pl.multiple_of(k * CH, 8)
            for j in range(CH // 16):
                srcb[pl.ds(j * 16, 16)] = src_st[pl.ds(off + j * 16, 16)]
                dstb[pl.ds(j * 16, 16)] = dst_st[pl.ds(off + j * 16, 16)]
            pltpu.async_copy(table_h.at[srcb], rows, sem).wait()
            pltpu.sync_copy(rows, acc.at[dstb], add=True)
            pltpu.sync_copy(onesb, cnt.at[dstb], add=True)
            return carry
        lax.fori_loop(0, NCHUNK, chunk, 0)
        plsc.subcore_barrier()

        pltpu.sync_copy(acc.at[pl.ds(rbase, ROWS_W)],
                        sum_out.at[c, pl.ds(rbase, ROWS_W)])
        pltpu.sync_copy(cnt.at[pl.ds(rbase, ROWS_W)],
                        cnt_out.at[c, pl.ds(rbase, ROWS_W)])
        plsc.subcore_barrier()

    run_edge_type(src_uv, dst_uv, pmsg_u, sum_v, cnt_v)
    run_edge_type(src_vu, dst_vu, pmsg_v, sum_u, cnt_u)


def _sc(pmsg_u, pmsg_v, src_uv, dst_uv, src_vu, dst_vu):
    f32 = jnp.float32
    mesh = plsc.VectorSubcoreMesh(core_axis_name="c", subcore_axis_name="s")
    k = functools.partial(
        pl.kernel,
        mesh=mesh,
        compiler_params=pltpu.CompilerParams(use_tc_tiling_on_sc=False),
        out_type=[
            jax.ShapeDtypeStruct((NC, N_PAD, H), f32),
            jax.ShapeDtypeStruct((NC, N_PAD, CNT_L), f32),
            jax.ShapeDtypeStruct((NC, N_PAD, H), f32),
            jax.ShapeDtypeStruct((NC, N_PAD, CNT_L), f32),
        ],
        scratch_types=[
            pltpu.VMEM((E_W,), jnp.int32),
            pltpu.VMEM((E_W,), jnp.int32),
            pltpu.VMEM((CH,), jnp.int32),
            pltpu.VMEM((CH,), jnp.int32),
            pltpu.VMEM((CH, H), f32),
            pltpu.VMEM((CH, CNT_L), f32),
            pltpu.VMEM((ROWS_W, H), f32),
            pltpu.VMEM((ROWS_W, CNT_L), f32),
            pltpu.VMEM_SHARED((N_PAD, H), f32),
            pltpu.VMEM_SHARED((N_PAD, CNT_L), f32),
            pltpu.SemaphoreType.DMA,
        ],
    )(_sc_body)
    return k(pmsg_u, pmsg_v, src_uv, dst_uv, src_vu, dst_vu)


def _post_body(su2, cu2, sv2, cv2, ru, rv, bvu, buv, wu, bu, wv, bv,
               outu, outv):
    sa = su2[...]
    ca = cu2[...]
    s_u = sa[0] + sa[1]
    c_u = (ca[0] + ca[1])[:, 0:1]
    mean_u = s_u / jnp.maximum(c_u, 1.0)
    h_u = jnp.maximum(mean_u + ru[...] + bvu[...], 0.0)
    outu[...] = jnp.dot(h_u, wu[...], preferred_element_type=jnp.float32) + bu[...]

    sb = sv2[...]
    cb = cv2[...]
    s_v = sb[0] + sb[1]
    c_v = (cb[0] + cb[1])[:, 0:1]
    mean_v = s_v / jnp.maximum(c_v, 1.0)
    h_v = jnp.maximum(mean_v + rv[...] + buv[...], 0.0)
    outv[...] = jnp.dot(h_v, wv[...], preferred_element_type=jnp.float32) + bv[...]


def _post(sum_u, cnt_u, sum_v, cnt_v, root_u, root_v,
          b_vu, b_uv, W_user, b_user, W_video, b_video):
    f32 = jnp.float32
    psum = pl.BlockSpec((NC, BL, H), lambda r: (0, r, 0))
    pcnt = pl.BlockSpec((NC, BL, CNT_L), lambda r: (0, r, 0))
    root = pl.BlockSpec((BL, H), lambda r: (r, 0))
    bias = pl.BlockSpec((1, H), lambda r: (0, 0))
    wsp = pl.BlockSpec((H, O), lambda r: (0, 0))
    bout = pl.BlockSpec((1, O), lambda r: (0, 0))
    out = pl.BlockSpec((BL, O), lambda r: (r, 0))
    return pl.pallas_call(
        _post_body,
        grid=(NBLK,),
        in_specs=[psum, pcnt, psum, pcnt, root, root, bias, bias,
                  wsp, bout, wsp, bout],
        out_specs=[out, out],
        out_shape=[jax.ShapeDtypeStruct((N, O), f32)] * 2,
    )(sum_u, cnt_u, sum_v, cnt_v, root_u, root_v, b_vu, b_uv,
      W_user, b_user, W_video, b_video)


def kernel(x_user, x_video, edge_index_uv, edge_index_vu,
           W_l_uv, W_r_uv, b_uv, W_l_vu, W_r_vu, b_vu,
           W_user, b_user, W_video, b_video):
    src_uv = edge_index_uv[0].astype(jnp.int32)
    dst_uv = edge_index_uv[1].astype(jnp.int32)
    src_vu = edge_index_vu[0].astype(jnp.int32)
    dst_vu = edge_index_vu[1].astype(jnp.int32)

    pmsg_u, pmsg_v, root_u, root_v = _pre(
        x_user, x_video, W_l_uv, W_r_vu, W_l_vu, W_r_uv)

    sum_v, cnt_v, sum_u, cnt_u = _sc(
        pmsg_u, pmsg_v, src_uv, dst_uv, src_vu, dst_vu)
    sum_v = sum_v[:, :N]
    cnt_v = cnt_v[:, :N]
    sum_u = sum_u[:, :N]
    cnt_u = cnt_u[:, :N]

    out_u, out_v = _post(
        sum_u, cnt_u, sum_v, cnt_v, root_u, root_v,
        b_vu.reshape(1, H), b_uv.reshape(1, H),
        W_user, b_user.reshape(1, O), W_video, b_video.reshape(1, O))
    return (out_u, out_v)


# speedup vs baseline: 10.8963x; 1.4366x over previous
import functools

import jax
import jax.numpy as jnp
from jax import lax
from jax.experimental import pallas as pl
from jax.experimental.pallas import tpu as pltpu
from jax.experimental.pallas import tpu_sc as plsc

N = 10000
E = 320000
D = 128
H = 64
O = 32

NC = 2
NS = 16
NW = NC * NS
E_W = E // NW
CH = 80
NCHUNK = E_W // CH
N_PAD = 10240
ROWS_W = N_PAD // NS
CNT_L = 16

BL = 1000
NBLK = N // BL


def _pre_body(xu, xv, wlu, wrv, wlv, wru, pu, pv, ru, rv):
    a = xu[...]
    b = xv[...]
    pu[...] = jnp.dot(a, wlu[...], preferred_element_type=jnp.float32)
    ru[...] = jnp.dot(a, wrv[...], preferred_element_type=jnp.float32)
    pv[...] = jnp.dot(b, wlv[...], preferred_element_type=jnp.float32)
    rv[...] = jnp.dot(b, wru[...], preferred_element_type=jnp.float32)


def _pre(x_user, x_video, W_l_uv, W_r_vu, W_l_vu, W_r_uv):
    row = pl.BlockSpec((BL, D), lambda r: (r, 0))
    wsp = pl.BlockSpec((D, H), lambda r: (0, 0))
    out = pl.BlockSpec((BL, H), lambda r: (r, 0))
    f32 = jnp.float32
    return pl.pallas_call(
        _pre_body,
        grid=(NBLK,),
        in_specs=[row, row, wsp, wsp, wsp, wsp],
        out_specs=[out, out, out, out],
        out_shape=[jax.ShapeDtypeStruct((N, H), f32)] * 4,
    )(x_user, x_video, W_l_uv, W_r_vu, W_l_vu, W_r_uv)


def _sc_body(pmsg_u, pmsg_v, src_uv, dst_uv, src_vu, dst_vu,
             sum_v, cnt_v, sum_u, cnt_u,
             src_st, dst_st, srcb0, dstb0, srcb1, dstb1, rows0, rows1,
             onesb, zcnt, acc, cnt, sem0, sem1):
    c = lax.axis_index("c")
    s = lax.axis_index("s")
    wid = c * NS + s
    ebase = wid * E_W
    rbase = s * ROWS_W

    zero16 = jnp.zeros((16,), jnp.float32)
    one16 = jnp.full((16,), 1.0, jnp.float32)

    def orow(i, carry):
        onesb[i, pl.ds(0, 16)] = one16
        zcnt[i, pl.ds(0, 16)] = zero16
        return carry
    lax.fori_loop(0, CH, orow, 0)

    def run_edge_type(src_h, dst_h, table_h, sum_out, cnt_out):
        def zrow(i, carry):
            for j in range(H // 16):
                rows0[i, pl.ds(j * 16, 16)] = zero16
            return carry
        lax.fori_loop(0, CH, zrow, 0)
        for j in range(ROWS_W // CH):
            pltpu.sync_copy(rows0, acc.at[pl.ds(rbase + j * CH, CH)])
            pltpu.sync_copy(zcnt, cnt.at[pl.ds(rbase + j * CH, CH)])
        pltpu.sync_copy(src_h.at[pl.ds(ebase, E_W)], src_st)
        pltpu.sync_copy(dst_h.at[pl.ds(ebase, E_W)], dst_st)
        plsc.subcore_barrier()

        def prep(kc, srcb_, dstb_):
            off = pl.multiple_of(kc * CH, 8)
            for j in range(CH // 16):
                srcb_[pl.ds(j * 16, 16)] = src_st[pl.ds(off + j * 16, 16)]
                dstb_[pl.ds(j * 16, 16)] = dst_st[pl.ds(off + j * 16, 16)]

        def fire(srcb_, rows_, sem_):
            pltpu.async_copy(table_h.at[srcb_], rows_, sem_)

        def wait(srcb_, rows_, sem_):
            pltpu.make_async_copy(table_h.at[srcb_], rows_, sem_).wait()

        def scat(rows_, dstb_):
            pltpu.sync_copy(rows_, acc.at[dstb_], add=True)
            pltpu.sync_copy(onesb, cnt.at[dstb_], add=True)

        prep(0, srcb0, dstb0)
        fire(srcb0, rows0, sem0)

        def chunk2(i, carry):
            k = 2 * i
            prep(k + 1, srcb1, dstb1)
            fire(srcb1, rows1, sem1)
            wait(srcb0, rows0, sem0)
            scat(rows0, dstb0)
            prep(k + 2, srcb0, dstb0)
            fire(srcb0, rows0, sem0)
            wait(srcb1, rows1, sem1)
            scat(rows1, dstb1)
            return carry
        lax.fori_loop(0, (NCHUNK - 1) // 2, chunk2, 0)
        wait(srcb0, rows0, sem0)
        scat(rows0, dstb0)
        plsc.subcore_barrier()

        pltpu.sync_copy(acc.at[pl.ds(rbase, ROWS_W)],
                        sum_out.at[c, pl.ds(rbase, ROWS_W)])
        pltpu.sync_copy(cnt.at[pl.ds(rbase, ROWS_W)],
                        cnt_out.at[c, pl.ds(rbase, ROWS_W)])
        plsc.subcore_barrier()

    run_edge_type(src_uv, dst_uv, pmsg_u, sum_v, cnt_v)
    run_edge_type(src_vu, dst_vu, pmsg_v, sum_u, cnt_u)


def _sc(pmsg_u, pmsg_v, src_uv, dst_uv, src_vu, dst_vu):
    f32 = jnp.float32
    mesh = plsc.VectorSubcoreMesh(core_axis_name="c", subcore_axis_name="s")
    k = functools.partial(
        pl.kernel,
        mesh=mesh,
        compiler_params=pltpu.CompilerParams(use_tc_tiling_on_sc=False),
        out_type=[
            jax.ShapeDtypeStruct((NC, N_PAD, H), f32),
            jax.ShapeDtypeStruct((NC, N_PAD, CNT_L), f32),
            jax.ShapeDtypeStruct((NC, N_PAD, H), f32),
            jax.ShapeDtypeStruct((NC, N_PAD, CNT_L), f32),
        ],
        scratch_types=[
            pltpu.VMEM((E_W,), jnp.int32),
            pltpu.VMEM((E_W,), jnp.int32),
            pltpu.VMEM((CH,), jnp.int32),
            pltpu.VMEM((CH,), jnp.int32),
            pltpu.VMEM((CH,), jnp.int32),
            pltpu.VMEM((CH,), jnp.int32),
            pltpu.VMEM((CH, H), f32),
            pltpu.VMEM((CH, H), f32),
            pltpu.VMEM((CH, CNT_L), f32),
            pltpu.VMEM((CH, CNT_L), f32),
            pltpu.VMEM_SHARED((N_PAD, H), f32),
            pltpu.VMEM_SHARED((N_PAD, CNT_L), f32),
            pltpu.SemaphoreType.DMA,
            pltpu.SemaphoreType.DMA,
        ],
    )(_sc_body)
    return k(pmsg_u, pmsg_v, src_uv, dst_uv, src_vu, dst_vu)


def _post_body(su2, cu2, sv2, cv2, ru, rv, bvu, buv, wu, bu, wv, bv,
               outu, outv):
    sa = su2[...]
    ca = cu2[...]
    s_u = sa[0] + sa[1]
    c_u = (ca[0] + ca[1])[:, 0:1]
    mean_u = s_u / jnp.maximum(c_u, 1.0)
    h_u = jnp.maximum(mean_u + ru[...] + bvu[...], 0.0)
    outu[...] = jnp.dot(h_u, wu[...], preferred_element_type=jnp.float32) + bu[...]

    sb = sv2[...]
    cb = cv2[...]
    s_v = sb[0] + sb[1]
    c_v = (cb[0] + cb[1])[:, 0:1]
    mean_v = s_v / jnp.maximum(c_v, 1.0)
    h_v = jnp.maximum(mean_v + rv[...] + buv[...], 0.0)
    outv[...] = jnp.dot(h_v, wv[...], preferred_element_type=jnp.float32) + bv[...]


def _post(sum_u, cnt_u, sum_v, cnt_v, root_u, root_v,
          b_vu, b_uv, W_user, b_user, W_video, b_video):
    f32 = jnp.float32
    psum = pl.BlockSpec((NC, BL, H), lambda r: (0, r, 0))
    pcnt = pl.BlockSpec((NC, BL, CNT_L), lambda r: (0, r, 0))
    root = pl.BlockSpec((BL, H), lambda r: (r, 0))
    bias = pl.BlockSpec((1, H), lambda r: (0, 0))
    wsp = pl.BlockSpec((H, O), lambda r: (0, 0))
    bout = pl.BlockSpec((1, O), lambda r: (0, 0))
    out = pl.BlockSpec((BL, O), lambda r: (r, 0))
    return pl.pallas_call(
        _post_body,
        grid=(NBLK,),
        in_specs=[psum, pcnt, psum, pcnt, root, root, bias, bias,
                  wsp, bout, wsp, bout],
        out_specs=[out, out],
        out_shape=[jax.ShapeDtypeStruct((N, O), f32)] * 2,
    )(sum_u, cnt_u, sum_v, cnt_v, root_u, root_v, b_vu, b_uv,
      W_user, b_user, W_video, b_video)


def kernel(x_user, x_video, edge_index_uv, edge_index_vu,
           W_l_uv, W_r_uv, b_uv, W_l_vu, W_r_vu, b_vu,
           W_user, b_user, W_video, b_video):
    src_uv = edge_index_uv[0].astype(jnp.int32)
    dst_uv = edge_index_uv[1].astype(jnp.int32)
    src_vu = edge_index_vu[0].astype(jnp.int32)
    dst_vu = edge_index_vu[1].astype(jnp.int32)

    pmsg_u, pmsg_v, root_u, root_v = _pre(
        x_user, x_video, W_l_uv, W_r_vu, W_l_vu, W_r_uv)

    sum_v, cnt_v, sum_u, cnt_u = _sc(
        pmsg_u, pmsg_v, src_uv, dst_uv, src_vu, dst_vu)
    sum_v = sum_v[:, :N]
    cnt_v = cnt_v[:, :N]
    sum_u = sum_u[:, :N]
    cnt_u = cnt_u[:, :N]

    out_u, out_v = _post(
        sum_u, cnt_u, sum_v, cnt_v, root_u, root_v,
        b_vu.reshape(1, H), b_uv.reshape(1, H),
        W_user, b_user.reshape(1, O), W_video, b_video.reshape(1, O))
    return (out_u, out_v)
